# baseline (speedup 1.0000x reference)
import functools

import jax
import jax.numpy as jnp
from jax import lax
from jax.experimental import pallas as pl
from jax.experimental.pallas import tpu as pltpu
from jax.experimental.pallas import tpu_sc as plsc

N = 10000
E = 320000
D_IN = 128
H = 32
D_OUT = 64

NPAD = 10240
CH = 128
TOTAL_CHUNKS = 2560
E_PAD = TOTAL_CHUNKS * CH
ROWS_PER_SUB = NPAD // 16

EDGE_CFG = {H: (128, 32, 8), D_OUT: (130, 30, 5)}
DEG_CPT0, DEG_CPT1 = 112, 48
WB = 5

_f32 = jnp.float32
_i32 = jnp.int32


def _zero_vmem_2d(ref, nrows, ncols):
    def row(i, _):
        for jj in range(ncols // 16):
            ref[i, pl.ds(jj * 16, 16)] = jnp.zeros((16,), _f32)
        return 0
    lax.fori_loop(0, nrows, row, 0)


def _zero_vmem_1d(ref, nwords):
    def blk(i, _):
        ref[pl.ds(i * 16, 16)] = jnp.zeros((16,), _f32)
        return 0
    lax.fori_loop(0, nwords // 16, blk, 0)


def _make_deg_kernel():
    mesh = plsc.VectorSubcoreMesh(core_axis_name="c", subcore_axis_name="s")

    @functools.partial(
        pl.kernel,
        out_type=jax.ShapeDtypeStruct((2, NPAD), _f32),
        mesh=mesh,
        scratch_types=[
            pltpu.VMEM((DEG_CPT0, CH), _i32),
            pltpu.VMEM((CH,), _f32),
            pltpu.VMEM((ROWS_PER_SUB,), _f32),
            pltpu.VMEM_SHARED((NPAD,), _f32),
            pltpu.SemaphoreType.DMA,
        ],
    )
    def deg_kernel(dst_hbm, out_hbm, dst_v, ones_v, stage_v, acc_sh, sem):
        cid = lax.axis_index("c")
        sid = lax.axis_index("s")

        for k in range(CH // 16):
            ones_v[pl.ds(k * 16, 16)] = jnp.ones((16,), _f32)
        _zero_vmem_1d(stage_v, ROWS_PER_SUB)

        pltpu.sync_copy(stage_v, acc_sh.at[pl.ds(sid * ROWS_PER_SUB, ROWS_PER_SUB)])

        def work(cpt, base):
            pltpu.sync_copy(dst_hbm.at[pl.ds(base, cpt)], dst_v.at[pl.ds(0, cpt)])
            plsc.subcore_barrier()

            def fire(c, _):
                pltpu.async_copy(ones_v, acc_sh.at[dst_v.at[c]], sem, add=True)
                return 0
            lax.fori_loop(0, cpt, fire, 0)

            def drain(c, _):
                pltpu.make_async_copy(ones_v, acc_sh.at[dst_v.at[0]], sem).wait()
                return 0
            lax.fori_loop(0, cpt, drain, 0)

        @pl.when(cid == 0)
        def _():
            work(DEG_CPT0, sid * DEG_CPT0)

        @pl.when(cid == 1)
        def _():
            work(DEG_CPT1, 16 * DEG_CPT0 + sid * DEG_CPT1)

        plsc.subcore_barrier()
        rpw = ROWS_PER_SUB // 5
        for k in range(5):
            r0 = sid * ROWS_PER_SUB + k * rpw
            pltpu.async_copy(acc_sh.at[pl.ds(r0, rpw)],
                             out_hbm.at[cid, pl.ds(r0, rpw)], sem)
        for k in range(5):
            r0 = sid * ROWS_PER_SUB + k * rpw
            pltpu.make_async_copy(acc_sh.at[pl.ds(r0, rpw)],
                                  out_hbm.at[cid, pl.ds(r0, rpw)], sem).wait()

    return deg_kernel


def _make_edge_kernel(D):
    CPT0, CPT1, G = EDGE_CFG[D]
    HALF0, HALF1 = CPT0 // 2, CPT1 // 2
    mesh = plsc.VectorSubcoreMesh(core_axis_name="c", subcore_axis_name="s")

    @functools.partial(
        pl.kernel,
        out_type=(jax.ShapeDtypeStruct((NPAD, D), _f32),
                  jax.ShapeDtypeStruct((NPAD, D // 2), _i32)),
        mesh=mesh,
        scratch_types=[
            pltpu.VMEM((HALF0, CH), _i32),
            pltpu.VMEM((HALF0, CH), _i32),
            pltpu.VMEM((G * CH, D), _f32),
            pltpu.VMEM((ROWS_PER_SUB, D // 2), _i32),
            pltpu.VMEM_SHARED((NPAD, D), _f32),
        ] + [pltpu.SemaphoreType.DMA] * G,
        compiler_params=pltpu.CompilerParams(use_tc_tiling_on_sc=False,
                                             needs_layout_passes=False),
    )
    def edge_kernel(xs_hbm, src_hbm, dst_hbm, out_hbm, outb_hbm,
                    src_v, dst_v, rows_v, rowsb_v, acc_sh, *sems):
        cid = lax.axis_index("c")
        sid = lax.axis_index("s")

        _zero_vmem_2d(rows_v, CH, D)
        for r in range(ROWS_PER_SUB // CH):
            pltpu.sync_copy(rows_v.at[pl.ds(0, CH)],
                            acc_sh.at[pl.ds(sid * ROWS_PER_SUB + r * CH, CH)])

        def work(half_n, base0):
            for half in range(2):
                base = base0 + half * half_n
                pltpu.sync_copy(src_hbm.at[pl.ds(base, half_n)],
                                src_v.at[pl.ds(0, half_n)])
                pltpu.sync_copy(dst_hbm.at[pl.ds(base, half_n)],
                                dst_v.at[pl.ds(0, half_n)])
                if half == 0:
                    plsc.subcore_barrier()

                for j in range(G):
                    pltpu.async_copy(xs_hbm.at[src_v.at[j]],
                                     rows_v.at[pl.ds(j * CH, CH)], sems[j])

                def group(g, _):
                    for j in range(G):
                        c = g * G + j
                        pltpu.make_async_copy(xs_hbm.at[src_v.at[c]],
                                              rows_v.at[pl.ds(j * CH, CH)],
                                              sems[j]).wait()
                        pltpu.sync_copy(rows_v.at[pl.ds(j * CH, CH)],
                                        acc_sh.at[dst_v.at[c]], add=True)

                        @pl.when(c + G < half_n)
                        def _():
                            pltpu.async_copy(xs_hbm.at[src_v.at[c + G]],
                                             rows_v.at[pl.ds(j * CH, CH)],
                                             sems[j])
                    return 0
                lax.fori_loop(0, half_n // G, group, 0)

        @pl.when(cid == 0)
        def _():
            work(HALF0, sid * CPT0)

        @pl.when(cid == 1)
        def _():
            work(HALF1, 16 * CPT0 + sid * CPT1)

        plsc.subcore_barrier()
        rpw = ROWS_PER_SUB // WB

        @pl.when(cid == 0)
        def _():
            for k in range(WB):
                r0 = sid * ROWS_PER_SUB + k * rpw
                pltpu.async_copy(acc_sh.at[pl.ds(r0, rpw)],
                                 out_hbm.at[pl.ds(r0, rpw)], sems[k])
            for k in range(WB):
                r0 = sid * ROWS_PER_SUB + k * rpw
                pltpu.make_async_copy(acc_sh.at[pl.ds(r0, rpw)],
                                      out_hbm.at[pl.ds(r0, rpw)],
                                      sems[k]).wait()

        @pl.when(cid == 1)
        def _():
            r0 = sid * ROWS_PER_SUB
            pltpu.sync_copy(acc_sh.at[pl.ds(r0, ROWS_PER_SUB)],
                            rows_v.at[pl.ds(0, ROWS_PER_SUB)])

            def rnd(v):
                bits = plsc.bitcast(v, _i32)
                return (bits + 0x7FFF + ((bits >> 16) & 1))

            def conv(r, _):
                for t in range(D // 32):
                    a = rnd(rows_v[r, pl.ds(32 * t, 16)])
                    b = rnd(rows_v[r, pl.ds(32 * t + 16, 16)])
                    w = (b & jnp.int32(-65536)) | lax.shift_right_logical(a, 16)
                    rowsb_v[r, pl.ds(16 * t, 16)] = w
                return 0
            lax.fori_loop(0, ROWS_PER_SUB, conv, 0)

            for k in range(WB):
                pltpu.async_copy(rowsb_v.at[pl.ds(k * rpw, rpw)],
                                 outb_hbm.at[pl.ds(r0 + k * rpw, rpw)],
                                 sems[k])
            for k in range(WB):
                pltpu.make_async_copy(rowsb_v.at[pl.ds(k * rpw, rpw)],
                                      outb_hbm.at[pl.ds(r0 + k * rpw, rpw)],
                                      sems[k]).wait()

    return edge_kernel


_deg_call = _make_deg_kernel()
_edge_call_1 = _make_edge_kernel(H)
_edge_call_2 = _make_edge_kernel(D_OUT)


_BR = 1000


def _mm1_body(x_ref, w_ref, o_ref):
    o_ref[...] = jnp.dot(x_ref[...], w_ref[...], preferred_element_type=_f32)


def _mm1(x, W1):
    return pl.pallas_call(
        _mm1_body,
        grid=(N // _BR,),
        in_specs=[pl.BlockSpec((_BR, D_IN), lambda i: (i, 0)),
                  pl.BlockSpec((D_IN, H), lambda i: (0, 0))],
        out_specs=pl.BlockSpec((_BR, H), lambda i: (i, 0)),
        out_shape=jax.ShapeDtypeStruct((N, H), _f32),
    )(x, W1)


def _scale_body(xw_ref, deg_ref, xs_ref, dinv_ref):
    d = deg_ref[0] + deg_ref[1] + 1.0
    dinv = lax.rsqrt(d)
    dinv_ref[...] = dinv
    xs_ref[...] = xw_ref[...] * dinv


def _scale(xw1, degp):
    return pl.pallas_call(
        _scale_body,
        grid=(N // _BR,),
        in_specs=[pl.BlockSpec((_BR, H), lambda i: (i, 0)),
                  pl.BlockSpec((2, _BR, 1), lambda i: (0, i, 0))],
        out_specs=[pl.BlockSpec((_BR, H), lambda i: (i, 0)),
                   pl.BlockSpec((_BR, 1), lambda i: (i, 0))],
        out_shape=[jax.ShapeDtypeStruct((N, H), _f32),
                   jax.ShapeDtypeStruct((N, 1), _f32)],
    )(xw1, degp)


def _unpack_bf16_pairs(w, D):
    lo = lax.bitcast_convert_type(lax.shift_left(w, 16), _f32)
    hi = lax.bitcast_convert_type(w & jnp.int32(-65536), _f32)
    parts = []
    for t in range(D // 32):
        parts += [lo[:, 16 * t:16 * t + 16], hi[:, 16 * t:16 * t + 16]]
    return jnp.concatenate(parts, axis=1)


def _mid_body(acc_ref, accb_ref, xs_ref, dinv_ref, b1_ref, w2_ref, o_ref):
    a = acc_ref[...] + _unpack_bf16_pairs(accb_ref[...], H) + xs_ref[...]
    pre = a * dinv_ref[...] + b1_ref[...]
    h = jnp.where(pre > 0.0, pre, jnp.exp(jnp.minimum(pre, 0.0)) - 1.0)
    o_ref[...] = jnp.dot(h, w2_ref[...], preferred_element_type=_f32) * dinv_ref[...]


def _mid(acc1, acc1b, xs1, dinv, b1, W2):
    return pl.pallas_call(
        _mid_body,
        grid=(N // _BR,),
        in_specs=[pl.BlockSpec((_BR, H), lambda i: (i, 0)),
                  pl.BlockSpec((_BR, H // 2), lambda i: (i, 0)),
                  pl.BlockSpec((_BR, H), lambda i: (i, 0)),
                  pl.BlockSpec((_BR, 1), lambda i: (i, 0)),
                  pl.BlockSpec((1, H), lambda i: (0, 0)),
                  pl.BlockSpec((H, D_OUT), lambda i: (0, 0))],
        out_specs=pl.BlockSpec((_BR, D_OUT), lambda i: (i, 0)),
        out_shape=jax.ShapeDtypeStruct((N, D_OUT), _f32),
    )(acc1, acc1b, xs1, dinv, b1, W2)


def _final_body(acc_ref, accb_ref, xs_ref, dinv_ref, b2_ref, wo_ref, bo_ref, o_ref):
    a = acc_ref[...] + _unpack_bf16_pairs(accb_ref[...], D_OUT) + xs_ref[...]
    t = a * dinv_ref[...] + b2_ref[...]
    logits = jnp.dot(t, wo_ref[...], preferred_element_type=_f32) + bo_ref[...]
    m = jnp.max(logits, axis=1, keepdims=True)
    lse = m + jnp.log(jnp.sum(jnp.exp(logits - m), axis=1, keepdims=True))
    o_ref[...] = logits - lse


def _final(acc2, acc2b, xs2, dinv, b2, Wo, bo):
    return pl.pallas_call(
        _final_body,
        grid=(N // _BR,),
        in_specs=[pl.BlockSpec((_BR, D_OUT), lambda i: (i, 0)),
                  pl.BlockSpec((_BR, D_OUT // 2), lambda i: (i, 0)),
                  pl.BlockSpec((_BR, D_OUT), lambda i: (i, 0)),
                  pl.BlockSpec((_BR, 1), lambda i: (i, 0)),
                  pl.BlockSpec((1, D_OUT), lambda i: (0, 0)),
                  pl.BlockSpec((D_OUT, 2), lambda i: (0, 0)),
                  pl.BlockSpec((1, 2), lambda i: (0, 0))],
        out_specs=pl.BlockSpec((_BR, 2), lambda i: (i, 0)),
        out_shape=jax.ShapeDtypeStruct((N, 2), _f32),
    )(acc2, acc2b, xs2, dinv, b2, Wo, bo)


def kernel(x, edge_index, W1, b1, W2, b2, Wo, bo):
    src = edge_index[0]
    dst = edge_index[1]
    pad = E_PAD - E
    srcp = jnp.concatenate([src, jnp.zeros((pad,), _i32)]).reshape(E_PAD // CH, CH)
    dstp = jnp.concatenate([dst, jnp.full((pad,), N, _i32)]).reshape(E_PAD // CH, CH)

    degp = _deg_call(dstp)
    xw1 = _mm1(x, W1)

    xs1, dinv = _scale(xw1, degp.reshape(2, NPAD, 1))

    acc1, acc1b = _edge_call_1(xs1, srcp, dstp)

    xs2 = _mid(acc1, acc1b, xs1, dinv, b1.reshape(1, H), W2)

    acc2, acc2b = _edge_call_2(xs2, srcp, dstp)

    return _final(acc2, acc2b, xs2, dinv, b2.reshape(1, D_OUT), Wo,
                  bo.reshape(1, 2))


# speedup vs baseline: 1.0555x; 1.0555x over previous
import functools

import jax
import jax.numpy as jnp
from jax import lax
from jax.experimental import pallas as pl
from jax.experimental.pallas import tpu as pltpu
from jax.experimental.pallas import tpu_sc as plsc

N = 10000
E = 320000
D_IN = 128
H = 32
D_OUT = 64

NPAD = 10240
CH = 128
TOTAL_CHUNKS = 2560
E_PAD = TOTAL_CHUNKS * CH
ROWS_PER_SUB = NPAD // 16

EDGE_CFG = {H: (140, 20, 7, 5), D_OUT: (140, 20, 5, 5)}
DEG_CPT0, DEG_CPT1 = 112, 48
WB = 5

_f32 = jnp.float32
_i32 = jnp.int32


def _zero_vmem_2d(ref, nrows, ncols):
    def row(i, _):
        for jj in range(ncols // 16):
            ref[i, pl.ds(jj * 16, 16)] = jnp.zeros((16,), _f32)
        return 0
    lax.fori_loop(0, nrows, row, 0)


def _zero_vmem_1d(ref, nwords):
    def blk(i, _):
        ref[pl.ds(i * 16, 16)] = jnp.zeros((16,), _f32)
        return 0
    lax.fori_loop(0, nwords // 16, blk, 0)


def _make_deg_kernel():
    mesh = plsc.VectorSubcoreMesh(core_axis_name="c", subcore_axis_name="s")

    @functools.partial(
        pl.kernel,
        out_type=jax.ShapeDtypeStruct((2, NPAD), _f32),
        mesh=mesh,
        scratch_types=[
            pltpu.VMEM((DEG_CPT0, CH), _i32),
            pltpu.VMEM((CH,), _f32),
            pltpu.VMEM((ROWS_PER_SUB,), _f32),
            pltpu.VMEM_SHARED((NPAD,), _f32),
            pltpu.SemaphoreType.DMA,
        ],
    )
    def deg_kernel(dst_hbm, out_hbm, dst_v, ones_v, stage_v, acc_sh, sem):
        cid = lax.axis_index("c")
        sid = lax.axis_index("s")

        for k in range(CH // 16):
            ones_v[pl.ds(k * 16, 16)] = jnp.ones((16,), _f32)
        _zero_vmem_1d(stage_v, ROWS_PER_SUB)

        pltpu.sync_copy(stage_v, acc_sh.at[pl.ds(sid * ROWS_PER_SUB, ROWS_PER_SUB)])

        def work(cpt, base):
            pltpu.sync_copy(dst_hbm.at[pl.ds(base, cpt)], dst_v.at[pl.ds(0, cpt)])
            plsc.subcore_barrier()

            def fire(c, _):
                pltpu.async_copy(ones_v, acc_sh.at[dst_v.at[c]], sem, add=True)
                return 0
            lax.fori_loop(0, cpt, fire, 0)

            def drain(c, _):
                pltpu.make_async_copy(ones_v, acc_sh.at[dst_v.at[0]], sem).wait()
                return 0
            lax.fori_loop(0, cpt, drain, 0)

        @pl.when(cid == 0)
        def _():
            work(DEG_CPT0, sid * DEG_CPT0)

        @pl.when(cid == 1)
        def _():
            work(DEG_CPT1, 16 * DEG_CPT0 + sid * DEG_CPT1)

        plsc.subcore_barrier()
        rpw = ROWS_PER_SUB // 5
        for k in range(5):
            r0 = sid * ROWS_PER_SUB + k * rpw
            pltpu.async_copy(acc_sh.at[pl.ds(r0, rpw)],
                             out_hbm.at[cid, pl.ds(r0, rpw)], sem)
        for k in range(5):
            r0 = sid * ROWS_PER_SUB + k * rpw
            pltpu.make_async_copy(acc_sh.at[pl.ds(r0, rpw)],
                                  out_hbm.at[cid, pl.ds(r0, rpw)], sem).wait()

    return deg_kernel


def _make_edge_kernel(D):
    CPT0, CPT1, G0, G1 = EDGE_CFG[D]
    HALF0 = CPT0 // 2
    G = max(G0, G1)
    mesh = plsc.VectorSubcoreMesh(core_axis_name="c", subcore_axis_name="s")

    @functools.partial(
        pl.kernel,
        out_type=(jax.ShapeDtypeStruct((NPAD, D), _f32),
                  jax.ShapeDtypeStruct((NPAD, D // 2), _i32)),
        mesh=mesh,
        scratch_types=[
            pltpu.VMEM((HALF0, CH), _i32),
            pltpu.VMEM((HALF0, CH), _i32),
            pltpu.VMEM((G * CH, D), _f32),
            pltpu.VMEM((ROWS_PER_SUB, D // 2), _i32),
            pltpu.VMEM_SHARED((NPAD, D), _f32),
        ] + [pltpu.SemaphoreType.DMA] * G,
        compiler_params=pltpu.CompilerParams(use_tc_tiling_on_sc=False,
                                             needs_layout_passes=False),
    )
    def edge_kernel(xs_hbm, src_hbm, dst_hbm, out_hbm, outb_hbm,
                    src_v, dst_v, rows_v, rowsb_v, acc_sh, *sems):
        cid = lax.axis_index("c")
        sid = lax.axis_index("s")

        _zero_vmem_2d(rows_v, CH, D)
        for r in range(ROWS_PER_SUB // CH):
            pltpu.sync_copy(rows_v.at[pl.ds(0, CH)],
                            acc_sh.at[pl.ds(sid * ROWS_PER_SUB + r * CH, CH)])

        def stage(n, base):
            pltpu.sync_copy(src_hbm.at[pl.ds(base, n)],
                            src_v.at[pl.ds(0, n)])
            pltpu.sync_copy(dst_hbm.at[pl.ds(base, n)],
                            dst_v.at[pl.ds(0, n)])

        def pipe(n, g_depth):
            for j in range(g_depth):
                pltpu.async_copy(xs_hbm.at[src_v.at[j]],
                                 rows_v.at[pl.ds(j * CH, CH)], sems[j])

            def group(g, _):
                for j in range(g_depth):
                    c = g * g_depth + j
                    pltpu.make_async_copy(xs_hbm.at[src_v.at[c]],
                                          rows_v.at[pl.ds(j * CH, CH)],
                                          sems[j]).wait()
                    pltpu.sync_copy(rows_v.at[pl.ds(j * CH, CH)],
                                    acc_sh.at[dst_v.at[c]], add=True)

                    @pl.when(c + g_depth < n)
                    def _():
                        pltpu.async_copy(xs_hbm.at[src_v.at[c + g_depth]],
                                         rows_v.at[pl.ds(j * CH, CH)],
                                         sems[j])
                return 0
            lax.fori_loop(0, n // g_depth, group, 0)

        @pl.when(cid == 0)
        def _():
            stage(HALF0, sid * CPT0)

        @pl.when(cid == 1)
        def _():
            stage(CPT1, 16 * CPT0 + sid * CPT1)

        plsc.subcore_barrier()

        @pl.when(cid == 0)
        def _():
            pipe(HALF0, G0)
            stage(HALF0, sid * CPT0 + HALF0)
            pipe(HALF0, G0)

        @pl.when(cid == 1)
        def _():
            pipe(CPT1, G1)

        plsc.subcore_barrier()
        rpw = ROWS_PER_SUB // WB

        @pl.when(cid == 0)
        def _():
            for k in range(WB):
                r0 = sid * ROWS_PER_SUB + k * rpw
                pltpu.async_copy(acc_sh.at[pl.ds(r0, rpw)],
                                 out_hbm.at[pl.ds(r0, rpw)], sems[k])
            for k in range(WB):
                r0 = sid * ROWS_PER_SUB + k * rpw
                pltpu.make_async_copy(acc_sh.at[pl.ds(r0, rpw)],
                                      out_hbm.at[pl.ds(r0, rpw)],
                                      sems[k]).wait()

        @pl.when(cid == 1)
        def _():
            r0 = sid * ROWS_PER_SUB
            pltpu.sync_copy(acc_sh.at[pl.ds(r0, ROWS_PER_SUB)],
                            rows_v.at[pl.ds(0, ROWS_PER_SUB)])

            def rnd(v):
                bits = plsc.bitcast(v, _i32)
                return (bits + 0x7FFF + ((bits >> 16) & 1))

            def conv(r, _):
                for t in range(D // 32):
                    a = rnd(rows_v[r, pl.ds(32 * t, 16)])
                    b = rnd(rows_v[r, pl.ds(32 * t + 16, 16)])
                    w = (b & jnp.int32(-65536)) | lax.shift_right_logical(a, 16)
                    rowsb_v[r, pl.ds(16 * t, 16)] = w
                return 0
            lax.fori_loop(0, ROWS_PER_SUB, conv, 0)

            for k in range(WB):
                pltpu.async_copy(rowsb_v.at[pl.ds(k * rpw, rpw)],
                                 outb_hbm.at[pl.ds(r0 + k * rpw, rpw)],
                                 sems[k])
            for k in range(WB):
                pltpu.make_async_copy(rowsb_v.at[pl.ds(k * rpw, rpw)],
                                      outb_hbm.at[pl.ds(r0 + k * rpw, rpw)],
                                      sems[k]).wait()

    return edge_kernel


_deg_call = _make_deg_kernel()
_edge_call_1 = _make_edge_kernel(H)
_edge_call_2 = _make_edge_kernel(D_OUT)


_BR = 1000


def _mm1_body(x_ref, w_ref, o_ref):
    o_ref[...] = jnp.dot(x_ref[...], w_ref[...], preferred_element_type=_f32)


def _mm1(x, W1):
    return pl.pallas_call(
        _mm1_body,
        grid=(N // _BR,),
        in_specs=[pl.BlockSpec((_BR, D_IN), lambda i: (i, 0)),
                  pl.BlockSpec((D_IN, H), lambda i: (0, 0))],
        out_specs=pl.BlockSpec((_BR, H), lambda i: (i, 0)),
        out_shape=jax.ShapeDtypeStruct((N, H), _f32),
    )(x, W1)


def _scale_body(xw_ref, deg_ref, xs_ref, dinv_ref):
    d = deg_ref[0] + deg_ref[1] + 1.0
    dinv = lax.rsqrt(d)
    dinv_ref[...] = dinv
    xs_ref[...] = xw_ref[...] * dinv


def _scale(xw1, degp):
    return pl.pallas_call(
        _scale_body,
        grid=(N // _BR,),
        in_specs=[pl.BlockSpec((_BR, H), lambda i: (i, 0)),
                  pl.BlockSpec((2, _BR, 1), lambda i: (0, i, 0))],
        out_specs=[pl.BlockSpec((_BR, H), lambda i: (i, 0)),
                   pl.BlockSpec((_BR, 1), lambda i: (i, 0))],
        out_shape=[jax.ShapeDtypeStruct((N, H), _f32),
                   jax.ShapeDtypeStruct((N, 1), _f32)],
    )(xw1, degp)


def _unpack_bf16_pairs(w, D):
    lo = lax.bitcast_convert_type(lax.shift_left(w, 16), _f32)
    hi = lax.bitcast_convert_type(w & jnp.int32(-65536), _f32)
    parts = []
    for t in range(D // 32):
        parts += [lo[:, 16 * t:16 * t + 16], hi[:, 16 * t:16 * t + 16]]
    return jnp.concatenate(parts, axis=1)


def _mid_body(acc_ref, accb_ref, xs_ref, dinv_ref, b1_ref, w2_ref, o_ref):
    a = acc_ref[...] + _unpack_bf16_pairs(accb_ref[...], H) + xs_ref[...]
    pre = a * dinv_ref[...] + b1_ref[...]
    h = jnp.where(pre > 0.0, pre, jnp.exp(jnp.minimum(pre, 0.0)) - 1.0)
    o_ref[...] = jnp.dot(h, w2_ref[...], preferred_element_type=_f32) * dinv_ref[...]


def _mid(acc1, acc1b, xs1, dinv, b1, W2):
    return pl.pallas_call(
        _mid_body,
        grid=(N // _BR,),
        in_specs=[pl.BlockSpec((_BR, H), lambda i: (i, 0)),
                  pl.BlockSpec((_BR, H // 2), lambda i: (i, 0)),
                  pl.BlockSpec((_BR, H), lambda i: (i, 0)),
                  pl.BlockSpec((_BR, 1), lambda i: (i, 0)),
                  pl.BlockSpec((1, H), lambda i: (0, 0)),
                  pl.BlockSpec((H, D_OUT), lambda i: (0, 0))],
        out_specs=pl.BlockSpec((_BR, D_OUT), lambda i: (i, 0)),
        out_shape=jax.ShapeDtypeStruct((N, D_OUT), _f32),
    )(acc1, acc1b, xs1, dinv, b1, W2)


def _final_body(acc_ref, accb_ref, xs_ref, dinv_ref, b2_ref, wo_ref, bo_ref, o_ref):
    a = acc_ref[...] + _unpack_bf16_pairs(accb_ref[...], D_OUT) + xs_ref[...]
    t = a * dinv_ref[...] + b2_ref[...]
    logits = jnp.dot(t, wo_ref[...], preferred_element_type=_f32) + bo_ref[...]
    m = jnp.max(logits, axis=1, keepdims=True)
    lse = m + jnp.log(jnp.sum(jnp.exp(logits - m), axis=1, keepdims=True))
    o_ref[...] = logits - lse


def _final(acc2, acc2b, xs2, dinv, b2, Wo, bo):
    return pl.pallas_call(
        _final_body,
        grid=(N // _BR,),
        in_specs=[pl.BlockSpec((_BR, D_OUT), lambda i: (i, 0)),
                  pl.BlockSpec((_BR, D_OUT // 2), lambda i: (i, 0)),
                  pl.BlockSpec((_BR, D_OUT), lambda i: (i, 0)),
                  pl.BlockSpec((_BR, 1), lambda i: (i, 0)),
                  pl.BlockSpec((1, D_OUT), lambda i: (0, 0)),
                  pl.BlockSpec((D_OUT, 2), lambda i: (0, 0)),
                  pl.BlockSpec((1, 2), lambda i: (0, 0))],
        out_specs=pl.BlockSpec((_BR, 2), lambda i: (i, 0)),
        out_shape=jax.ShapeDtypeStruct((N, 2), _f32),
    )(acc2, acc2b, xs2, dinv, b2, Wo, bo)


def kernel(x, edge_index, W1, b1, W2, b2, Wo, bo):
    src = edge_index[0]
    dst = edge_index[1]
    pad = E_PAD - E
    srcp = jnp.concatenate([src, jnp.zeros((pad,), _i32)]).reshape(E_PAD // CH, CH)
    dstp = jnp.concatenate([dst, jnp.full((pad,), N, _i32)]).reshape(E_PAD // CH, CH)

    degp = _deg_call(dstp)
    xw1 = _mm1(x, W1)

    xs1, dinv = _scale(xw1, degp.reshape(2, NPAD, 1))

    acc1, acc1b = _edge_call_1(xs1, srcp, dstp)

    xs2 = _mid(acc1, acc1b, xs1, dinv, b1.reshape(1, H), W2)

    acc2, acc2b = _edge_call_2(xs2, srcp, dstp)

    return _final(acc2, acc2b, xs2, dinv, b2.reshape(1, D_OUT), Wo,
                  bo.reshape(1, 2))
